# baseline (speedup 1.0000x reference)
import functools

import jax
import jax.numpy as jnp
from jax import lax
from jax.experimental import pallas as pl
from jax.experimental.pallas import tpu as pltpu
from jax.experimental.pallas import tpu_sc as plsc

R = 16384
C = 4096
G = 128
L = 16
NC = 2
NS = 16
NW = NC * NS

SROWS = 8192
ROWS_PER_W = SROWS // NW
KR = 8
CHUNKS = ROWS_PER_W // KR
SLOT = KR * C

TROWS = R - SROWS
BR = 512


def _sc_body(x_hbm, inds_hbm, out_hbm, inds_v, idxt0, idxt1, gbuf,
             fsem0, fsem1, gsem0, gsem1, osem0, osem1, spmem):
    idxts = (idxt0, idxt1)
    fsems = (fsem0, fsem1)
    gsems = (gsem0, gsem1)
    osems = (osem0, osem1)
    cid = lax.axis_index("c")
    sid = lax.axis_index("s")
    wid = sid * NC + cid
    row0 = wid * ROWS_PER_W

    pltpu.sync_copy(inds_hbm, inds_v)
    for b in range(2):
        roff = (sid * 2 + b) * SLOT
        for j in range(KR):
            for t in range(G // L):
                iv = inds_v[pl.ds(t * L, L)]
                idxts[b][pl.ds(j * G + t * L, L)] = roff + j * C + iv

    def fire_fill(c, b):
        roff = (sid * 2 + b) * SLOT
        base_row = row0 + c * KR
        for j in range(KR):
            pltpu.async_copy(
                x_hbm.at[base_row + j],
                spmem.at[pl.ds(roff + j * C, C)], fsems[b],
            )

    def drain_fill(b):
        roff = (sid * 2 + b) * SLOT
        for j in range(KR):
            pltpu.make_async_copy(
                x_hbm.at[0], spmem.at[pl.ds(roff + j * C, C)], fsems[b]
            ).wait()

    def run_gather(b):
        for j in range(KR):
            pltpu.async_copy(
                spmem.at[idxts[b].at[pl.ds(j * G, G)]],
                gbuf.at[b].at[j], gsems[b],
            )
        for j in range(KR):
            pltpu.make_async_copy(
                spmem.at[idxts[b].at[pl.ds(j * G, G)]],
                gbuf.at[b].at[j], gsems[b],
            ).wait()

    def fire_out(c, b):
        base_row = row0 + c * KR
        pltpu.async_copy(
            gbuf.at[b], out_hbm.at[pl.ds(base_row, KR)], osems[b]
        )

    def drain_out(b):
        pltpu.make_async_copy(
            gbuf.at[b], out_hbm.at[pl.ds(0, KR)], osems[b]
        ).wait()

    fire_fill(0, 0)
    fire_fill(1, 1)

    for b in range(2):
        drain_fill(b)
        run_gather(b)
        fire_fill(b + 2, b)
        fire_out(b, b)

    def group(gi, carry):
        for b in range(2):
            c = 2 * gi + b
            drain_fill(b)
            drain_out(b)
            run_gather(b)
            fire_fill(c + 2, b)
            fire_out(c, b)
        return carry

    lax.fori_loop(1, CHUNKS // 2 - 1, group, 0)

    for b in range(2):
        c = CHUNKS - 2 + b
        drain_fill(b)
        drain_out(b)
        run_gather(b)
        fire_out(c, b)
    for b in range(2):
        drain_out(b)


def _sc_gather(x, inds_flat):
    mesh = plsc.VectorSubcoreMesh(core_axis_name="c", subcore_axis_name="s")
    run = functools.partial(
        pl.kernel,
        mesh=mesh,
        compiler_params=pltpu.CompilerParams(
            needs_layout_passes=False, use_tc_tiling_on_sc=True
        ),
        cost_estimate=pl.CostEstimate(
            flops=0,
            bytes_accessed=SROWS * C * 4 + SROWS * G * 8,
            transcendentals=0,
        ),
        out_type=jax.ShapeDtypeStruct((SROWS, G), jnp.float32),
        scratch_types=[
            pltpu.VMEM((G,), jnp.int32),
            pltpu.VMEM((KR * G,), jnp.int32),
            pltpu.VMEM((KR * G,), jnp.int32),
            pltpu.VMEM((2, KR, G), jnp.float32),
            pltpu.SemaphoreType.DMA,
            pltpu.SemaphoreType.DMA,
            pltpu.SemaphoreType.DMA,
            pltpu.SemaphoreType.DMA,
            pltpu.SemaphoreType.DMA,
            pltpu.SemaphoreType.DMA,
            pltpu.VMEM_SHARED((NS * 2 * SLOT,), jnp.float32),
        ],
    )(_sc_body)
    return run(x, inds_flat)


def _mm_body(inds_ref, x_ref, o_ref, sel_ref):
    @pl.when(pl.program_id(0) == 0)
    def _():
        iota = lax.broadcasted_iota(jnp.int32, (C, G), 0)
        sel_ref[...] = (iota == inds_ref[0, :][None, :]).astype(jnp.float32)

    o_ref[...] = jnp.dot(
        x_ref[...], sel_ref[...], preferred_element_type=jnp.float32
    )


def _tc_gather(x_full, inds32):
    return pl.pallas_call(
        _mm_body,
        grid=(TROWS // BR,),
        in_specs=[
            pl.BlockSpec((1, G), lambda i: (0, 0)),
            pl.BlockSpec((BR, C), lambda i: (i + SROWS // BR, 0)),
        ],
        out_specs=pl.BlockSpec((BR, G), lambda i: (i, 0)),
        out_shape=jax.ShapeDtypeStruct((TROWS, G), jnp.float32),
        scratch_shapes=[pltpu.VMEM((C, G), jnp.float32)],
        cost_estimate=pl.CostEstimate(
            flops=2 * TROWS * C * G,
            bytes_accessed=TROWS * C * 4 + TROWS * G * 4,
            transcendentals=0,
        ),
    )(inds32, x_full)


@jax.jit
def kernel(x, inds):
    inds32 = inds.astype(jnp.int32)
    sc_out = _sc_gather(x, inds32.reshape(G))
    tc_out = _tc_gather(x, inds32)
    return jnp.concatenate([sc_out, tc_out], axis=0)


# speedup vs baseline: 1.0796x; 1.0796x over previous
import functools

import jax
import jax.numpy as jnp
from jax import lax
from jax.experimental import pallas as pl
from jax.experimental.pallas import tpu as pltpu
from jax.experimental.pallas import tpu_sc as plsc

R = 16384
C = 4096
G = 128
L = 16
NC = 2
NS = 16
NW = NC * NS

SROWS = 6144
ROWS_PER_W = SROWS // NW
KR = 8
CHUNKS = ROWS_PER_W // KR
SLOT = KR * C

TROWS = R - SROWS
BR = 512


def _sc_body(x_hbm, inds_hbm, out_hbm, inds_v, idxt0, idxt1, gbuf,
             fsem0, fsem1, gsem0, gsem1, osem0, osem1, spmem):
    idxts = (idxt0, idxt1)
    fsems = (fsem0, fsem1)
    gsems = (gsem0, gsem1)
    osems = (osem0, osem1)
    cid = lax.axis_index("c")
    sid = lax.axis_index("s")
    wid = sid * NC + cid
    row0 = wid * ROWS_PER_W

    pltpu.sync_copy(inds_hbm, inds_v)
    for b in range(2):
        roff = (sid * 2 + b) * SLOT
        for j in range(KR):
            for t in range(G // L):
                iv = inds_v[pl.ds(t * L, L)]
                idxts[b][pl.ds(j * G + t * L, L)] = roff + j * C + iv

    def fire_fill(c, b):
        roff = (sid * 2 + b) * SLOT
        base_row = row0 + c * KR
        for j in range(KR):
            pltpu.async_copy(
                x_hbm.at[base_row + j],
                spmem.at[pl.ds(roff + j * C, C)], fsems[b],
            )

    def drain_fill(b):
        roff = (sid * 2 + b) * SLOT
        for j in range(KR):
            pltpu.make_async_copy(
                x_hbm.at[0], spmem.at[pl.ds(roff + j * C, C)], fsems[b]
            ).wait()

    def run_gather(b):
        for j in range(KR):
            pltpu.async_copy(
                spmem.at[idxts[b].at[pl.ds(j * G, G)]],
                gbuf.at[b].at[j], gsems[b],
            )
        for j in range(KR):
            pltpu.make_async_copy(
                spmem.at[idxts[b].at[pl.ds(j * G, G)]],
                gbuf.at[b].at[j], gsems[b],
            ).wait()

    def fire_out(c, b):
        base_row = row0 + c * KR
        pltpu.async_copy(
            gbuf.at[b], out_hbm.at[pl.ds(base_row, KR)], osems[b]
        )

    def drain_out(b):
        pltpu.make_async_copy(
            gbuf.at[b], out_hbm.at[pl.ds(0, KR)], osems[b]
        ).wait()

    fire_fill(0, 0)
    fire_fill(1, 1)

    for b in range(2):
        drain_fill(b)
        run_gather(b)
        fire_fill(b + 2, b)
        fire_out(b, b)

    def group(gi, carry):
        for b in range(2):
            c = 2 * gi + b
            drain_fill(b)
            drain_out(b)
            run_gather(b)
            fire_fill(c + 2, b)
            fire_out(c, b)
        return carry

    lax.fori_loop(1, CHUNKS // 2 - 1, group, 0)

    for b in range(2):
        c = CHUNKS - 2 + b
        drain_fill(b)
        drain_out(b)
        run_gather(b)
        fire_out(c, b)
    for b in range(2):
        drain_out(b)


def _sc_gather(x, inds_flat):
    mesh = plsc.VectorSubcoreMesh(core_axis_name="c", subcore_axis_name="s")
    run = functools.partial(
        pl.kernel,
        mesh=mesh,
        compiler_params=pltpu.CompilerParams(
            needs_layout_passes=False, use_tc_tiling_on_sc=True
        ),
        cost_estimate=pl.CostEstimate(
            flops=0,
            bytes_accessed=SROWS * C * 4 + SROWS * G * 8,
            transcendentals=0,
        ),
        out_type=jax.ShapeDtypeStruct((SROWS, G), jnp.float32),
        scratch_types=[
            pltpu.VMEM((G,), jnp.int32),
            pltpu.VMEM((KR * G,), jnp.int32),
            pltpu.VMEM((KR * G,), jnp.int32),
            pltpu.VMEM((2, KR, G), jnp.float32),
            pltpu.SemaphoreType.DMA,
            pltpu.SemaphoreType.DMA,
            pltpu.SemaphoreType.DMA,
            pltpu.SemaphoreType.DMA,
            pltpu.SemaphoreType.DMA,
            pltpu.SemaphoreType.DMA,
            pltpu.VMEM_SHARED((NS * 2 * SLOT,), jnp.float32),
        ],
    )(_sc_body)
    return run(x, inds_flat)


def _mm_body(inds_ref, x_ref, o_ref, sel_ref):
    @pl.when(pl.program_id(0) == 0)
    def _():
        iota = lax.broadcasted_iota(jnp.int32, (C, G), 0)
        sel_ref[...] = (iota == inds_ref[0, :][None, :]).astype(jnp.float32)

    o_ref[...] = jnp.dot(
        x_ref[...], sel_ref[...], preferred_element_type=jnp.float32
    )


def _tc_gather(x_full, inds32):
    return pl.pallas_call(
        _mm_body,
        grid=(TROWS // BR,),
        in_specs=[
            pl.BlockSpec((1, G), lambda i: (0, 0)),
            pl.BlockSpec((BR, C), lambda i: (i + SROWS // BR, 0)),
        ],
        out_specs=pl.BlockSpec((BR, G), lambda i: (i, 0)),
        out_shape=jax.ShapeDtypeStruct((TROWS, G), jnp.float32),
        scratch_shapes=[pltpu.VMEM((C, G), jnp.float32)],
        cost_estimate=pl.CostEstimate(
            flops=2 * TROWS * C * G,
            bytes_accessed=TROWS * C * 4 + TROWS * G * 4,
            transcendentals=0,
        ),
    )(inds32, x_full)


@jax.jit
def kernel(x, inds):
    inds32 = inds.astype(jnp.int32)
    sc_out = _sc_gather(x, inds32.reshape(G))
    tc_out = _tc_gather(x, inds32)
    return jnp.concatenate([sc_out, tc_out], axis=0)
